# Initial kernel scaffold; baseline (speedup 1.0000x reference)
import jax
import jax.numpy as jnp
from jax.experimental import pallas as pl


def kernel(x, edge_index, Wl0, bl0, Wr0, Wl1, bl1, Wr1, g0, be0, g1, be1, W1, b1, W2, b2):
    raise NotImplementedError("write your pallas kernel here")


# speedup vs baseline: 2.6659x; 2.6659x over previous
import functools

import jax
import jax.numpy as jnp
from jax import lax
from jax.experimental import pallas as pl
from jax.experimental.pallas import tpu as pltpu
from jax.experimental.pallas import tpu_sc as plsc

_N = 10000
_E = 160000
_K = 128
_NT = 16
_NCH = 80
_EPAD = _NT * _NCH * _K
_NPAD = 10112
_ZPT = _NPAD // _NT


@functools.lru_cache(maxsize=None)
def _make_sc_agg(nblocks):
    nbpc = nblocks // 2
    mesh = plsc.VectorSubcoreMesh(core_axis_name="c", subcore_axis_name="s",
                                  num_cores=2, num_subcores=16)

    @functools.partial(
        pl.kernel,
        mesh=mesh,
        out_type=jax.ShapeDtypeStruct((nblocks, _NPAD, 128), jnp.float32),
        scratch_types=[
            pltpu.VMEM((_NCH, _K), jnp.int32),
            pltpu.VMEM((_NCH, _K), jnp.int32),
            pltpu.VMEM((_K, 128), jnp.float32),
            pltpu.VMEM_SHARED((_NPAD, 128), jnp.float32),
            pltpu.SemaphoreType.DMA,
        ],
    )
    def agg(table, srcoff, dstc, zeros, out, src_v, dst_v, rows_v, acc, sem):
        c = lax.axis_index("c")
        s = lax.axis_index("s")
        pltpu.sync_copy(dstc.at[s], dst_v)
        for ib in range(nbpc):
            b = c * nbpc + ib
            pltpu.sync_copy(srcoff.at[b, s], src_v)
            pltpu.sync_copy(zeros.at[pl.ds(s * _ZPT, _ZPT)],
                            acc.at[pl.ds(s * _ZPT, _ZPT)])
            plsc.subcore_barrier()

            def chunk(j, carry):
                pltpu.async_copy(table.at[src_v.at[j]], rows_v, sem).wait()
                pltpu.sync_copy(rows_v, acc.at[dst_v.at[j]], add=True)
                return carry

            lax.fori_loop(0, _NCH, chunk, 0)
            plsc.subcore_barrier()
            pltpu.sync_copy(acc.at[pl.ds(s * _ZPT, _ZPT)],
                            out.at[b, pl.ds(s * _ZPT, _ZPT)])
            plsc.subcore_barrier()

    return agg


_TILE = 1000
_GRID = _N // _TILE


def _sage_stats_body(nblocks, x_ref, agg_ref, wl_ref, wr_ref, b_ref,
                     y_ref, st_ref):
    i = pl.program_id(0)
    acc = jnp.dot(x_ref[...], wr_ref[...], preferred_element_type=jnp.float32)
    for cb in range(nblocks):
        acc += jnp.dot(agg_ref[cb], wl_ref[cb],
                       preferred_element_type=jnp.float32)
    acc += b_ref[...]
    y_ref[...] = acc
    s = jnp.sum(acc, axis=0, keepdims=True)
    q = jnp.sum(acc * acc, axis=0, keepdims=True)
    st = jnp.concatenate([s, q, jnp.zeros((6, 512), jnp.float32)], axis=0)

    @pl.when(i == 0)
    def _():
        st_ref[...] = st

    @pl.when(i > 0)
    def _():
        st_ref[...] += st


def _sage_layer(nblocks, din, x, agg_blocks, wlT_blocks, wrT, bl):
    return pl.pallas_call(
        functools.partial(_sage_stats_body, nblocks),
        grid=(_GRID,),
        in_specs=[
            pl.BlockSpec((_TILE, din), lambda i: (i, 0)),
            pl.BlockSpec((nblocks, _TILE, 128), lambda i: (0, i, 0)),
            pl.BlockSpec((nblocks, 128, 512), lambda i: (0, 0, 0)),
            pl.BlockSpec((din, 512), lambda i: (0, 0)),
            pl.BlockSpec((1, 512), lambda i: (0, 0)),
        ],
        out_specs=[
            pl.BlockSpec((_TILE, 512), lambda i: (i, 0)),
            pl.BlockSpec((8, 512), lambda i: (0, 0)),
        ],
        out_shape=[
            jax.ShapeDtypeStruct((_N, 512), jnp.float32),
            jax.ShapeDtypeStruct((8, 512), jnp.float32),
        ],
    )(x, agg_blocks, wlT_blocks, wrT, bl)


def _bn_relu_body(y_ref, st_ref, g_ref, be_ref, h_ref):
    inv_n = 1.0 / _N
    mean = st_ref[0:1, :] * inv_n
    var = st_ref[1:2, :] * inv_n - mean * mean
    scale = g_ref[...] * lax.rsqrt(var + 1e-5)
    shift = be_ref[...] - mean * scale
    h_ref[...] = jnp.maximum(y_ref[...] * scale + shift, 0.0)


def _bn_relu(y, st, g, be):
    return pl.pallas_call(
        _bn_relu_body,
        grid=(_GRID,),
        in_specs=[
            pl.BlockSpec((_TILE, 512), lambda i: (i, 0)),
            pl.BlockSpec((8, 512), lambda i: (0, 0)),
            pl.BlockSpec((1, 512), lambda i: (0, 0)),
            pl.BlockSpec((1, 512), lambda i: (0, 0)),
        ],
        out_specs=pl.BlockSpec((_TILE, 512), lambda i: (i, 0)),
        out_shape=jax.ShapeDtypeStruct((_N, 512), jnp.float32),
    )(y, st, g, be)


def _head_body(y_ref, st_ref, g_ref, be_ref, w1_ref, b1_ref, w2_ref, b2_ref,
               o_ref):
    inv_n = 1.0 / _N
    mean = st_ref[0:1, :] * inv_n
    var = st_ref[1:2, :] * inv_n - mean * mean
    scale = g_ref[...] * lax.rsqrt(var + 1e-5)
    shift = be_ref[...] - mean * scale
    h1 = jnp.maximum(y_ref[...] * scale + shift, 0.0)
    h2 = jnp.dot(h1, w1_ref[...], preferred_element_type=jnp.float32)
    h2 = jnp.maximum(h2 + b1_ref[...], 0.0)
    o = jnp.dot(h2, w2_ref[...], preferred_element_type=jnp.float32)
    o_ref[...] = o + b2_ref[...]


def _head(y, st, g, be, w1T, b1, w2T, b2):
    return pl.pallas_call(
        _head_body,
        grid=(_GRID,),
        in_specs=[
            pl.BlockSpec((_TILE, 512), lambda i: (i, 0)),
            pl.BlockSpec((8, 512), lambda i: (0, 0)),
            pl.BlockSpec((1, 512), lambda i: (0, 0)),
            pl.BlockSpec((1, 512), lambda i: (0, 0)),
            pl.BlockSpec((512, 512), lambda i: (0, 0)),
            pl.BlockSpec((1, 512), lambda i: (0, 0)),
            pl.BlockSpec((512, 256), lambda i: (0, 0)),
            pl.BlockSpec((1, 256), lambda i: (0, 0)),
        ],
        out_specs=pl.BlockSpec((_TILE, 256), lambda i: (i, 0)),
        out_shape=jax.ShapeDtypeStruct((_N, 256), jnp.float32),
    )(y, st, g, be, w1T, b1, w2T, b2)


def kernel(x, edge_index, Wl0, bl0, Wr0, Wl1, bl1, Wr1, g0, be0, g1, be1,
           W1, b1, W2, b2):
    src = edge_index[0]
    dst = edge_index[1]
    pad = _EPAD - _E
    src_p = jnp.concatenate([src, jnp.zeros((pad,), jnp.int32)])
    dst_p = jnp.concatenate([dst, jnp.full((pad,), _N, jnp.int32)])
    dst_t = dst_p.reshape(_NT, _NCH, _K)
    off2 = jnp.arange(2, dtype=jnp.int32)[:, None]
    off4 = jnp.arange(4, dtype=jnp.int32)[:, None]
    srcoff2 = (src_p[None, :] * 2 + off2).reshape(2, _NT, _NCH, _K)
    srcoff4 = (src_p[None, :] * 4 + off4).reshape(4, _NT, _NCH, _K)
    zeros = jnp.zeros((_NPAD, 128), jnp.float32)

    wl0T = Wl0.T.reshape(2, 128, 512)
    wr0T = Wr0.T
    wl1T = Wl1.T.reshape(4, 128, 512)
    wr1T = Wr1.T
    w1T = W1.T
    w2T = W2.T
    bl0r = bl0.reshape(1, 512)
    bl1r = bl1.reshape(1, 512)
    g0r = g0.reshape(1, 512)
    be0r = be0.reshape(1, 512)
    g1r = g1.reshape(1, 512)
    be1r = be1.reshape(1, 512)
    b1r = b1.reshape(1, 512)
    b2r = b2.reshape(1, 256)

    agg0 = _make_sc_agg(2)(x.reshape(2 * _N, 128), srcoff2, dst_t, zeros)
    y0, st0 = _sage_layer(2, 256, x, agg0, wl0T, wr0T, bl0r)
    h0 = _bn_relu(y0, st0, g0r, be0r)

    agg1 = _make_sc_agg(4)(h0.reshape(4 * _N, 128), srcoff4, dst_t, zeros)
    y1, st1 = _sage_layer(4, 512, h0, agg1, wl1T, wr1T, bl1r)

    return _head(y1, st1, g1r, be1r, w1T, b1r, w2T, b2r)


# speedup vs baseline: 3.2483x; 1.2185x over previous
import functools

import jax
import jax.numpy as jnp
from jax import lax
from jax.experimental import pallas as pl
from jax.experimental.pallas import tpu as pltpu
from jax.experimental.pallas import tpu_sc as plsc

_N = 10000
_E = 160000
_K = 64
_NT = 16
_NCH = 160
_NSR = 2
_HCH = _NCH // _NSR
_EPAD = _NT * _NCH * _K
_NPAD = 10112
_ZPT = _NPAD // _NT


@functools.lru_cache(maxsize=None)
def _make_sc_agg(nblocks):
    nbpc = nblocks // 2
    mesh = plsc.VectorSubcoreMesh(core_axis_name="c", subcore_axis_name="s",
                                  num_cores=2, num_subcores=16)

    @functools.partial(
        pl.kernel,
        mesh=mesh,
        out_type=jax.ShapeDtypeStruct((nblocks, _NPAD, 128), jnp.float32),
        scratch_types=[
            pltpu.VMEM((_HCH, _K), jnp.int32),
            pltpu.VMEM((_HCH, _K), jnp.int32),
            pltpu.VMEM((_K, 128), jnp.float32),
            pltpu.VMEM((_K, 128), jnp.float32),
            pltpu.VMEM_SHARED((_NPAD, 128), jnp.float32),
            pltpu.SemaphoreType.DMA,
            pltpu.SemaphoreType.DMA,
        ],
    )
    def agg(table, srcoff, dstc, zeros, out,
            src_v, dst_v, rows_a, rows_b, acc, sem_a, sem_b):
        c = lax.axis_index("c")
        s = lax.axis_index("s")
        for ib in range(nbpc):
            b = c * nbpc + ib
            pltpu.sync_copy(zeros.at[pl.ds(s * _ZPT, _ZPT)],
                            acc.at[pl.ds(s * _ZPT, _ZPT)])
            plsc.subcore_barrier()

            for sr in range(_NSR):
                pltpu.sync_copy(srcoff.at[b, s, pl.ds(sr * _HCH, _HCH)],
                                src_v)
                pltpu.sync_copy(dstc.at[s, pl.ds(sr * _HCH, _HCH)], dst_v)

                pltpu.async_copy(table.at[src_v.at[0]], rows_a, sem_a)

                def pair(i, carry):
                    j0 = 2 * i
                    j1 = j0 + 1
                    jn = jnp.minimum(j1 + 1, _HCH - 1)
                    pltpu.async_copy(table.at[src_v.at[j1]], rows_b, sem_b)
                    pltpu.make_async_copy(table.at[src_v.at[j0]],
                                          rows_a, sem_a).wait()
                    pltpu.sync_copy(rows_a, acc.at[dst_v.at[j0]], add=True)
                    pltpu.async_copy(table.at[src_v.at[jn]], rows_a, sem_a)
                    pltpu.make_async_copy(table.at[src_v.at[j1]],
                                          rows_b, sem_b).wait()
                    pltpu.sync_copy(rows_b, acc.at[dst_v.at[j1]], add=True)
                    return carry

                lax.fori_loop(0, _HCH // 2, pair, 0)
                pltpu.make_async_copy(table.at[src_v.at[0]],
                                      rows_a, sem_a).wait()
            plsc.subcore_barrier()
            pltpu.sync_copy(acc.at[pl.ds(s * _ZPT, _ZPT)],
                            out.at[b, pl.ds(s * _ZPT, _ZPT)])
            plsc.subcore_barrier()

    return agg


_TILE = 1000
_GRID = _N // _TILE


def _sage_stats_body(nblocks, x_ref, agg_ref, wl_ref, wr_ref, b_ref,
                     y_ref, st_ref):
    i = pl.program_id(0)
    acc = jnp.dot(x_ref[...], wr_ref[...], preferred_element_type=jnp.float32)
    for cb in range(nblocks):
        acc += jnp.dot(agg_ref[cb], wl_ref[cb],
                       preferred_element_type=jnp.float32)
    acc += b_ref[...]
    y_ref[...] = acc
    s = jnp.sum(acc, axis=0, keepdims=True)
    q = jnp.sum(acc * acc, axis=0, keepdims=True)
    st = jnp.concatenate([s, q, jnp.zeros((6, 512), jnp.float32)], axis=0)

    @pl.when(i == 0)
    def _():
        st_ref[...] = st

    @pl.when(i > 0)
    def _():
        st_ref[...] += st


def _sage_layer(nblocks, din, x, agg_blocks, wlT_blocks, wrT, bl):
    return pl.pallas_call(
        functools.partial(_sage_stats_body, nblocks),
        grid=(_GRID,),
        in_specs=[
            pl.BlockSpec((_TILE, din), lambda i: (i, 0)),
            pl.BlockSpec((nblocks, _TILE, 128), lambda i: (0, i, 0)),
            pl.BlockSpec((nblocks, 128, 512), lambda i: (0, 0, 0)),
            pl.BlockSpec((din, 512), lambda i: (0, 0)),
            pl.BlockSpec((1, 512), lambda i: (0, 0)),
        ],
        out_specs=[
            pl.BlockSpec((_TILE, 512), lambda i: (i, 0)),
            pl.BlockSpec((8, 512), lambda i: (0, 0)),
        ],
        out_shape=[
            jax.ShapeDtypeStruct((_N, 512), jnp.float32),
            jax.ShapeDtypeStruct((8, 512), jnp.float32),
        ],
    )(x, agg_blocks, wlT_blocks, wrT, bl)


def _bn_relu_body(y_ref, st_ref, g_ref, be_ref, h_ref):
    inv_n = 1.0 / _N
    mean = st_ref[0:1, :] * inv_n
    var = st_ref[1:2, :] * inv_n - mean * mean
    scale = g_ref[...] * lax.rsqrt(var + 1e-5)
    shift = be_ref[...] - mean * scale
    h_ref[...] = jnp.maximum(y_ref[...] * scale + shift, 0.0)


def _bn_relu(y, st, g, be):
    return pl.pallas_call(
        _bn_relu_body,
        grid=(_GRID,),
        in_specs=[
            pl.BlockSpec((_TILE, 512), lambda i: (i, 0)),
            pl.BlockSpec((8, 512), lambda i: (0, 0)),
            pl.BlockSpec((1, 512), lambda i: (0, 0)),
            pl.BlockSpec((1, 512), lambda i: (0, 0)),
        ],
        out_specs=pl.BlockSpec((_TILE, 512), lambda i: (i, 0)),
        out_shape=jax.ShapeDtypeStruct((_N, 512), jnp.float32),
    )(y, st, g, be)


def _head_body(y_ref, st_ref, g_ref, be_ref, w1_ref, b1_ref, w2_ref, b2_ref,
               o_ref):
    inv_n = 1.0 / _N
    mean = st_ref[0:1, :] * inv_n
    var = st_ref[1:2, :] * inv_n - mean * mean
    scale = g_ref[...] * lax.rsqrt(var + 1e-5)
    shift = be_ref[...] - mean * scale
    h1 = jnp.maximum(y_ref[...] * scale + shift, 0.0)
    h2 = jnp.dot(h1, w1_ref[...], preferred_element_type=jnp.float32)
    h2 = jnp.maximum(h2 + b1_ref[...], 0.0)
    o = jnp.dot(h2, w2_ref[...], preferred_element_type=jnp.float32)
    o_ref[...] = o + b2_ref[...]


def _head(y, st, g, be, w1T, b1, w2T, b2):
    return pl.pallas_call(
        _head_body,
        grid=(_GRID,),
        in_specs=[
            pl.BlockSpec((_TILE, 512), lambda i: (i, 0)),
            pl.BlockSpec((8, 512), lambda i: (0, 0)),
            pl.BlockSpec((1, 512), lambda i: (0, 0)),
            pl.BlockSpec((1, 512), lambda i: (0, 0)),
            pl.BlockSpec((512, 512), lambda i: (0, 0)),
            pl.BlockSpec((1, 512), lambda i: (0, 0)),
            pl.BlockSpec((512, 256), lambda i: (0, 0)),
            pl.BlockSpec((1, 256), lambda i: (0, 0)),
        ],
        out_specs=pl.BlockSpec((_TILE, 256), lambda i: (i, 0)),
        out_shape=jax.ShapeDtypeStruct((_N, 256), jnp.float32),
    )(y, st, g, be, w1T, b1, w2T, b2)


def kernel(x, edge_index, Wl0, bl0, Wr0, Wl1, bl1, Wr1, g0, be0, g1, be1,
           W1, b1, W2, b2):
    src = edge_index[0]
    dst = edge_index[1]
    pad = _EPAD - _E
    src_p = jnp.concatenate([src, jnp.zeros((pad,), jnp.int32)])
    dst_p = jnp.concatenate([dst, jnp.full((pad,), _N, jnp.int32)])
    dst_t = dst_p.reshape(_NT, _NCH, _K)
    off2 = jnp.arange(2, dtype=jnp.int32)[:, None]
    off4 = jnp.arange(4, dtype=jnp.int32)[:, None]
    srcoff2 = (src_p[None, :] * 2 + off2).reshape(2, _NT, _NCH, _K)
    srcoff4 = (src_p[None, :] * 4 + off4).reshape(4, _NT, _NCH, _K)
    zeros = jnp.zeros((_NPAD, 128), jnp.float32)

    wl0T = Wl0.T.reshape(2, 128, 512)
    wr0T = Wr0.T
    wl1T = Wl1.T.reshape(4, 128, 512)
    wr1T = Wr1.T
    w1T = W1.T
    w2T = W2.T
    bl0r = bl0.reshape(1, 512)
    bl1r = bl1.reshape(1, 512)
    g0r = g0.reshape(1, 512)
    be0r = be0.reshape(1, 512)
    g1r = g1.reshape(1, 512)
    be1r = be1.reshape(1, 512)
    b1r = b1.reshape(1, 512)
    b2r = b2.reshape(1, 256)

    agg0 = _make_sc_agg(2)(x.reshape(2 * _N, 128), srcoff2, dst_t, zeros)
    y0, st0 = _sage_layer(2, 256, x, agg0, wl0T, wr0T, bl0r)
    h0 = _bn_relu(y0, st0, g0r, be0r)

    agg1 = _make_sc_agg(4)(h0.reshape(4 * _N, 128), srcoff4, dst_t, zeros)
    y1, st1 = _sage_layer(4, 512, h0, agg1, wl1T, wr1T, bl1r)

    return _head(y1, st1, g1r, be1r, w1T, b1r, w2T, b2r)
